# baseline (speedup 1.0000x reference)
import functools
import math

import jax
import jax.numpy as jnp
from jax import lax
from jax.experimental import pallas as pl
from jax.experimental.pallas import tpu as pltpu
from jax.experimental.pallas import tpu_sc as plsc

_V = 1024
_D = 256
_NF = 10
_W = _D + 2 * _NF
_B = 16384
_F = 3
_NW = 32
_COLS_W = _B // _NW
_C = 128
_NCHUNK = _COLS_W // _C
_NSLAB = 34


def _fourier(rows):
    d = lax.broadcasted_iota(jnp.int32, (rows, 2 * _NF), 0).astype(jnp.float32)
    k = lax.broadcasted_iota(jnp.int32, (rows, 2 * _NF), 1)
    kk = k % _NF
    coef = (jnp.float32(math.pi) / jnp.float32(_V)) * (
        lax.shift_left(jnp.int32(1), kk).astype(jnp.float32))
    raw = coef * d
    return jnp.where(k < _NF, jnp.sin(raw), jnp.cos(raw))


def _tables_body(emb_ref, tb_ref, tc_ref):
    four = _fourier(_V)
    tb_ref[...] = jnp.concatenate(
        [jnp.zeros((_V, 2 * _NF), jnp.float32), emb_ref[:, :_D - 2 * _NF]],
        axis=1)
    tc_ref[...] = jnp.concatenate(
        [emb_ref[:, _D - 2 * _NF:], four,
         jnp.zeros((_V, 128 - 4 * _NF), jnp.float32)], axis=1)


def _build_tables(embed_table):
    return pl.pallas_call(
        _tables_body,
        out_shape=(
            jax.ShapeDtypeStruct((_V, _D), jnp.float32),
            jax.ShapeDtypeStruct((_V, 128), jnp.float32),
        ),
    )(embed_table)


def _sc_body(emb, tb, tc, t_T, out, t_v, idx_e, idx_o, rbufE, rbufB,
             rbufC, rbufD, slabbuf, half, gsems, ssem, hsem):
    wid = lax.axis_index("s") * 2 + lax.axis_index("c")
    pltpu.sync_copy(t_T.at[:, pl.ds(wid * _COLS_W, _COLS_W)], t_v)

    lane = lax.iota(jnp.int32, 16)
    zero = lane * 0
    for u in range(_COLS_W // 16):
        off16 = u * 16
        t0 = t_v[0, pl.ds(off16, 16)]
        t1 = t_v[1, pl.ds(off16, 16)]
        t2 = t_v[2, pl.ds(off16, 16)]
        cc = u // (_C // 16)
        off = (u % (_C // 16)) * 16
        idx_e[cc, pl.ds(off, 16)] = t1 - t0
        idx_o[cc, pl.ds(off, 16)] = t2 - t1

    col_base = wid * _COLS_W

    def _slab_view(s):
        blk = pl.multiple_of((s % 4) * 16, 16)
        return slabbuf.at[pl.ds(blk, 16), :]

    def _out_view(s, b0):
        row = pl.multiple_of(s * 16, 16)
        return out.at[pl.ds(row, 16), pl.ds(b0, _C)]

    def _strip(src, src_off, s):
        blk = (s % 4) * 16
        rows = blk + lane

        def grp(g, _):
            for i in range(8):
                b = g * 8 + i
                v = src[b, pl.ds(src_off, 16)]
                plsc.store_scatter(slabbuf, [rows, zero + b], v)
            return 0
        lax.fori_loop(0, _C // 8, grp, 0)

    def _strip_masked(src, src_off, dst, row_base, row_shift, mask_ge):
        def grp(g, _):
            for i in range(8):
                b = g * 8 + i
                v = src[b, pl.ds(src_off, 16)]
                plsc.store_scatter(
                    dst, [row_base + lane - row_shift, zero + b], v,
                    mask=lane >= mask_ge)
            return 0
        lax.fori_loop(0, _C // 8, grp, 0)

    def _wait_slab(b0):
        pltpu.make_async_copy(
            slabbuf.at[pl.ds(0, 16), :],
            out.at[pl.ds(0, 16), pl.ds(b0, _C)], ssem).wait()

    def _flush(s, b0):
        pltpu.async_copy(_slab_view(s), _out_view(s, b0), ssem)

    for c in range(_NCHUNK):
        b0 = pl.multiple_of(col_base + c * _C, _C)
        ga = pltpu.async_copy(emb.at[idx_e.at[c]], rbufE, gsems[0])
        gb = pltpu.async_copy(tb.at[idx_o.at[c]], rbufB, gsems[1])
        gc = pltpu.async_copy(tc.at[idx_e.at[c]], rbufC, gsems[2])
        gd = pltpu.async_copy(tc.at[idx_o.at[c]], rbufD, gsems[3])

        if c > 0:
            for _ in range(4):
                _wait_slab(b0)

        ga.wait()
        for s in range(4):
            _strip(rbufE, s * 16, s)
            _flush(s, b0)

        def mid_e(s, _):
            _wait_slab(b0)
            _strip(rbufE, s * 16, s)
            _flush(s, b0)
            return 0
        lax.fori_loop(4, 16, mid_e, 0)

        gc.wait()
        _wait_slab(b0)
        _strip(rbufC, 20, 16)
        _flush(16, b0)
        gb.wait()
        _wait_slab(b0)
        _strip(rbufB, 16, 17)
        _strip_masked(rbufC, 24, slabbuf, 16, 12, 12)
        _flush(17, b0)

        def mid_b(s, _):
            _wait_slab(b0)
            _strip(rbufB, (s - 16) * 16, s)
            _flush(s, b0)
            return 0
        lax.fori_loop(18, 32, mid_b, 0)

        gd.wait()
        _wait_slab(b0)
        _strip(rbufD, 0, 32)
        _flush(32, b0)
        _wait_slab(b0)
        _strip(rbufD, 16, 33)
        _flush(33, b0)

        if c > 0:
            pltpu.make_async_copy(
                half, out.at[pl.ds(544, 8), pl.ds(b0, _C)], hsem).wait()
        _strip_masked(rbufD, 24, half, 0, 8, 8)
        pltpu.async_copy(half, out.at[pl.ds(544, 8), pl.ds(b0, _C)], hsem)

    b0 = pl.multiple_of(col_base, _C)
    for _ in range(4):
        _wait_slab(b0)
    pltpu.make_async_copy(
        half, out.at[pl.ds(544, 8), pl.ds(b0, _C)], hsem).wait()


@functools.partial(
    pl.kernel,
    out_type=jax.ShapeDtypeStruct((2 * _W, _B), jnp.float32),
    mesh=plsc.VectorSubcoreMesh(core_axis_name="c", subcore_axis_name="s"),
    compiler_params=pltpu.CompilerParams(needs_layout_passes=False),
    scratch_types=[
        pltpu.VMEM((_F, _COLS_W), jnp.int32),
        pltpu.VMEM((_NCHUNK, _C), jnp.int32),
        pltpu.VMEM((_NCHUNK, _C), jnp.int32),
        pltpu.VMEM((_C, _D), jnp.float32),
        pltpu.VMEM((_C, _D), jnp.float32),
        pltpu.VMEM((_C, 128), jnp.float32),
        pltpu.VMEM((_C, 128), jnp.float32),
        pltpu.VMEM((64, 128), jnp.float32),
        pltpu.VMEM((8, 128), jnp.float32),
        pltpu.SemaphoreType.DMA,
        pltpu.SemaphoreType.DMA,
        pltpu.SemaphoreType.DMA,
        pltpu.SemaphoreType.DMA,
        pltpu.SemaphoreType.DMA,
        pltpu.SemaphoreType.DMA,
    ],
)
def _sc_gather(emb, tb, tc, t_T, out, t_v, idx_e, idx_o, rbufE, rbufB,
               rbufC, rbufD, slabbuf, half, g0, g1, g2, g3, ss, hs):
    _sc_body(emb, tb, tc, t_T, out, t_v, idx_e, idx_o, rbufE, rbufB,
             rbufC, rbufD, slabbuf, half, (g0, g1, g2, g3), ss, hs)


def kernel(t, embed_table):
    tb, tc = _build_tables(embed_table)
    return _sc_gather(embed_table, tb, tc, t.T).T


# speedup vs baseline: 2.0177x; 2.0177x over previous
import functools
import math

import jax
import jax.numpy as jnp
from jax import lax
from jax.experimental import pallas as pl
from jax.experimental.pallas import tpu as pltpu
from jax.experimental.pallas import tpu_sc as plsc

_V = 1024
_D = 256
_NF = 10
_W = _D + 2 * _NF
_B = 16384
_F = 3
_NW = 32
_ROWS_W = _B // _NW
_R = 64
_NCHUNK = _ROWS_W // _R
_NBUF = 2
_RT = 4 * _NF


def _fourier(rows):
    d = lax.broadcasted_iota(jnp.int32, (rows, 2 * _NF), 0).astype(jnp.float32)
    k = lax.broadcasted_iota(jnp.int32, (rows, 2 * _NF), 1)
    kk = k % _NF
    coef = (jnp.float32(math.pi) / jnp.float32(_V)) * (
        lax.shift_left(jnp.int32(1), kk).astype(jnp.float32))
    raw = coef * d
    return jnp.where(k < _NF, jnp.sin(raw), jnp.cos(raw))


def _tables_body(emb_ref, tb_ref, four_ref):
    tb_ref[...] = jnp.concatenate(
        [jnp.zeros((_V, 2 * _NF), jnp.float32), emb_ref[:, :_D - 2 * _NF]],
        axis=1)
    four_ref[...] = _fourier(_V)


def _build_tables(embed_table):
    return pl.pallas_call(
        _tables_body,
        out_shape=(
            jax.ShapeDtypeStruct((_V, _D), jnp.float32),
            jax.ShapeDtypeStruct((_V, 2 * _NF), jnp.float32),
        ),
    )(embed_table)


def _sc_body(emb, tb, rtab, t_T, out, t_v, idx_e, idx_o, rtab_v,
             obufs, sas, sbs, sos):
    wid = lax.axis_index("s") * 2 + lax.axis_index("c")
    pltpu.sync_copy(t_T.at[:, pl.ds(wid * _ROWS_W, _ROWS_W)], t_v)
    pltpu.sync_copy(rtab, rtab_v)

    lane = lax.iota(jnp.int32, 16)
    zero = lane * 0
    for u in range(_ROWS_W // 16):
        off16 = u * 16
        t0 = t_v[0, pl.ds(off16, 16)]
        t1 = t_v[1, pl.ds(off16, 16)]
        t2 = t_v[2, pl.ds(off16, 16)]
        cc = u // (_R // 16)
        off = (u % (_R // 16)) * 16
        idx_e[cc, pl.ds(off, 16)] = t1 - t0
        idx_o[cc, pl.ds(off, 16)] = t2 - t1

    orow_base = wid * _ROWS_W

    def _fire(c):
        p = c % _NBUF
        ga = pltpu.async_copy(
            emb.at[idx_e.at[c]], obufs[p].at[:, pl.ds(0, _D)], sas[p])
        gb = pltpu.async_copy(
            tb.at[idx_o.at[c]], obufs[p].at[:, pl.ds(_D, _D)], sbs[p])
        return (ga, gb)

    def _repair(c):
        p = c % _NBUF
        obuf = obufs[p]
        for s in range(_R // 16):
            rows = lane + (s * 16)
            f0 = idx_e[c, pl.ds(s * 16, 16)] * _RT + (2 * _NF)
            f1 = idx_o[c, pl.ds(s * 16, 16)] * _RT

            def f_fix(k4, _):
                k = k4 * 4
                for j in range(4):
                    fl = f0 + (k + j)
                    vals = plsc.load_gather(rtab_v, [fl >> 7, fl & 127])
                    plsc.store_scatter(
                        obuf, [rows, zero + (_D + k + j)], vals)
                return 0

            def t_fix(k4, _):
                k = k4 * 4
                for j in range(4):
                    fl = f1 + (k + j)
                    vals = plsc.load_gather(rtab_v, [fl >> 7, fl & 127])
                    plsc.store_scatter(
                        obuf, [rows, zero + (2 * _D + k + j)], vals)
                return 0

            lax.fori_loop(0, 2 * _NF // 4, f_fix, 0)
            lax.fori_loop(0, _RT // 4, t_fix, 0)

    gh = [None] * _NCHUNK
    oh = [None] * _NCHUNK
    for c in range(_NBUF - 1):
        gh[c] = _fire(c)
    for c in range(_NCHUNK):
        if c + _NBUF - 1 < _NCHUNK:
            if c >= 1:
                oh[c - 1].wait()
            gh[c + _NBUF - 1] = _fire(c + _NBUF - 1)
        for h in gh[c]:
            h.wait()
        _repair(c)
        p = c % _NBUF
        oh[c] = pltpu.make_async_copy(
            obufs[p], out.at[pl.ds(orow_base + c * _R, _R)], sos[p])
        oh[c].start()
    oh[_NCHUNK - 2].wait()
    oh[_NCHUNK - 1].wait()


@functools.partial(
    pl.kernel,
    out_type=jax.ShapeDtypeStruct((_B, 2 * _W), jnp.float32),
    mesh=plsc.VectorSubcoreMesh(core_axis_name="c", subcore_axis_name="s"),
    compiler_params=pltpu.CompilerParams(needs_layout_passes=False),
    scratch_types=[
        pltpu.VMEM((_F, _ROWS_W), jnp.int32),
        pltpu.VMEM((_NCHUNK, _R), jnp.int32),
        pltpu.VMEM((_NCHUNK, _R), jnp.int32),
        pltpu.VMEM((_V * _RT // 128, 128), jnp.float32),
        pltpu.VMEM((_R, 2 * _W), jnp.float32),
        pltpu.VMEM((_R, 2 * _W), jnp.float32),
        pltpu.SemaphoreType.DMA,
        pltpu.SemaphoreType.DMA,
        pltpu.SemaphoreType.DMA,
        pltpu.SemaphoreType.DMA,
        pltpu.SemaphoreType.DMA,
        pltpu.SemaphoreType.DMA,
    ],
)
def _sc_gather(emb, tb, rtab, t_T, out, t_v, idx_e, idx_o, rtab_v,
               ob0, ob1, a0, a1, b0, b1, o0, o1):
    _sc_body(emb, tb, rtab, t_T, out, t_v, idx_e, idx_o, rtab_v,
             (ob0, ob1), (a0, a1), (b0, b1), (o0, o1))


def kernel(t, embed_table):
    tb, four = _build_tables(embed_table)
    rtab = jnp.concatenate(
        [embed_table[:, _D - 2 * _NF:], four], axis=1).reshape(
            _V * _RT // 128, 128)
    return _sc_gather(embed_table, tb, rtab, t.T)


# speedup vs baseline: 2.0854x; 1.0335x over previous
import functools
import math

import jax
import jax.numpy as jnp
from jax import lax
from jax.experimental import pallas as pl
from jax.experimental.pallas import tpu as pltpu
from jax.experimental.pallas import tpu_sc as plsc

_V = 1024
_D = 256
_NF = 10
_W = _D + 2 * _NF
_B = 16384
_F = 3
_NW = 32
_ROWS_W = _B // _NW
_R = 64
_NCHUNK = _ROWS_W // _R
_NBUF = 2
_RT = 4 * _NF


def _fourier(rows):
    d = lax.broadcasted_iota(jnp.int32, (rows, 2 * _NF), 0).astype(jnp.float32)
    k = lax.broadcasted_iota(jnp.int32, (rows, 2 * _NF), 1)
    kk = k % _NF
    coef = (jnp.float32(math.pi) / jnp.float32(_V)) * (
        lax.shift_left(jnp.int32(1), kk).astype(jnp.float32))
    raw = coef * d
    return jnp.where(k < _NF, jnp.sin(raw), jnp.cos(raw))


def _tables_body(emb_ref, tb_ref, four_ref):
    tb_ref[...] = jnp.concatenate(
        [jnp.zeros((_V, 2 * _NF), jnp.float32), emb_ref[:, :_D - 2 * _NF]],
        axis=1)
    four_ref[...] = _fourier(_V)


def _build_tables(embed_table):
    return pl.pallas_call(
        _tables_body,
        out_shape=(
            jax.ShapeDtypeStruct((_V, _D), jnp.float32),
            jax.ShapeDtypeStruct((_V, 2 * _NF), jnp.float32),
        ),
    )(embed_table)


def _sc_body(emb, tb, rtab, t_T, out, t_v, idx_e, idx_o, rtab_v,
             obufs, sas, sbs, sos):
    wid = lax.axis_index("s") * 2 + lax.axis_index("c")
    pltpu.sync_copy(t_T.at[:, pl.ds(wid * _ROWS_W, _ROWS_W)], t_v)
    pltpu.sync_copy(rtab, rtab_v)

    lane = lax.iota(jnp.int32, 16)
    zero = lane * 0
    for u in range(_ROWS_W // 16):
        off16 = u * 16
        t0 = t_v[0, pl.ds(off16, 16)]
        t1 = t_v[1, pl.ds(off16, 16)]
        t2 = t_v[2, pl.ds(off16, 16)]
        cc = u // (_R // 16)
        off = (u % (_R // 16)) * 16
        idx_e[cc, pl.ds(off, 16)] = t1 - t0
        idx_o[cc, pl.ds(off, 16)] = t2 - t1

    orow_base = wid * _ROWS_W

    def _fire(c):
        p = c % _NBUF
        ga = pltpu.async_copy(
            emb.at[idx_e.at[c]], obufs[p].at[:, pl.ds(0, _D)], sas[p])
        gb = pltpu.async_copy(
            tb.at[idx_o.at[c]], obufs[p].at[:, pl.ds(_D, _D)], sbs[p])
        return (ga, gb)

    def _repair(c):
        p = c % _NBUF
        obuf = obufs[p]
        for s in range(_R // 16):
            rows = lane + (s * 16)
            f0 = idx_e[c, pl.ds(s * 16, 16)] * _RT + (2 * _NF)
            f1 = idx_o[c, pl.ds(s * 16, 16)] * _RT

            def f_fix(k, _):
                fl = f0 + k
                vals = plsc.load_gather(rtab_v, [fl >> 7, fl & 127])
                plsc.store_scatter(obuf, [rows, zero + (_D + k)], vals)
                return 0

            def t_fix(k, _):
                fl = f1 + k
                vals = plsc.load_gather(rtab_v, [fl >> 7, fl & 127])
                plsc.store_scatter(obuf, [rows, zero + (2 * _D + k)], vals)
                return 0

            lax.fori_loop(0, 2 * _NF, f_fix, 0)
            lax.fori_loop(0, _RT, t_fix, 0)

    gh = [None] * _NCHUNK
    oh = [None] * _NCHUNK
    for c in range(_NBUF - 1):
        gh[c] = _fire(c)
    for c in range(_NCHUNK):
        if c + _NBUF - 1 < _NCHUNK:
            if c >= 1:
                oh[c - 1].wait()
            gh[c + _NBUF - 1] = _fire(c + _NBUF - 1)
        for h in gh[c]:
            h.wait()
        _repair(c)
        p = c % _NBUF
        oh[c] = pltpu.make_async_copy(
            obufs[p], out.at[pl.ds(orow_base + c * _R, _R)], sos[p])
        oh[c].start()
    oh[_NCHUNK - 2].wait()
    oh[_NCHUNK - 1].wait()


@functools.partial(
    pl.kernel,
    out_type=jax.ShapeDtypeStruct((_B, 2 * _W), jnp.float32),
    mesh=plsc.VectorSubcoreMesh(core_axis_name="c", subcore_axis_name="s"),
    compiler_params=pltpu.CompilerParams(needs_layout_passes=False),
    scratch_types=[
        pltpu.VMEM((_F, _ROWS_W), jnp.int32),
        pltpu.VMEM((_NCHUNK, _R), jnp.int32),
        pltpu.VMEM((_NCHUNK, _R), jnp.int32),
        pltpu.VMEM((_V * _RT // 128, 128), jnp.float32),
        pltpu.VMEM((_R, 2 * _W), jnp.float32),
        pltpu.VMEM((_R, 2 * _W), jnp.float32),
        pltpu.SemaphoreType.DMA,
        pltpu.SemaphoreType.DMA,
        pltpu.SemaphoreType.DMA,
        pltpu.SemaphoreType.DMA,
        pltpu.SemaphoreType.DMA,
        pltpu.SemaphoreType.DMA,
    ],
)
def _sc_gather(emb, tb, rtab, t_T, out, t_v, idx_e, idx_o, rtab_v,
               ob0, ob1, a0, a1, b0, b1, o0, o1):
    _sc_body(emb, tb, rtab, t_T, out, t_v, idx_e, idx_o, rtab_v,
             (ob0, ob1), (a0, a1), (b0, b1), (o0, o1))


def kernel(t, embed_table):
    tb, four = _build_tables(embed_table)
    rtab = jnp.concatenate(
        [embed_table[:, _D - 2 * _NF:], four], axis=1).reshape(
            _V * _RT // 128, 128)
    return _sc_gather(embed_table, tb, rtab, t.T)
